# Initial kernel scaffold; baseline (speedup 1.0000x reference)
import jax
import jax.numpy as jnp
from jax.experimental import pallas as pl


def kernel(input, coeffs, G_rows, G_cols, G_vals, L_rows, L_cols, L_vals, F_rows, F_cols, F_vals, NS, EW):
    raise NotImplementedError("write your pallas kernel here")


# speedup vs baseline: 41.0464x; 41.0464x over previous
import functools

import jax
import jax.numpy as jnp
from jax import lax
from jax.experimental import pallas as pl
from jax.experimental.pallas import tpu as pltpu
from jax.experimental.pallas import tpu_sc as plsc

NV = 40962
NV_PREV = 10242
NF = 81920
BS = 2
C = 128
D = BS * C

_NC, _NSC = 2, 16
NW = _NC * _NSC
NVP = 41472
VPW = NVP // NW

FCH = 8
FNCH = (NF // NW) // FCH
LCH = 16
LNCH = VPW // LCH
VCH = 8
VNCH = VPW // VCH

_MESH = dict(core_axis_name="c", subcore_axis_name="s",
             num_cores=_NC, num_subcores=_NSC)


def _wid():
    return lax.axis_index("s") * _NC + lax.axis_index("c")


def _face_body(xt, gidx, gw, ewns, faces2, idx_v, gw_v, ewns_v, gbuf, obuf, sem):
    c0 = _wid() * FNCH

    def chunk(ci, carry):
        c = c0 + ci
        pltpu.sync_copy(gidx.at[c], idx_v)
        pltpu.sync_copy(gw.at[c], gw_v)
        pltpu.sync_copy(ewns.at[c], ewns_v)
        pltpu.async_copy(xt.at[idx_v], gbuf, sem).wait()
        for fb in range(FCH):
            g0 = fb * 9
            wb = min(g0, FCH * 9 - 16)
            eb = min(fb * 6, FCH * 6 - 16)
            wv = gw_v[pl.ds(wb, 16)]
            ev = ewns_v[pl.ds(eb, 16)]
            w = [wv[g0 - wb + j] for j in range(9)]
            e = [ev[fb * 6 - eb + j] for j in range(6)]
            for t in range(D // 16):
                sl = pl.ds(t * 16, 16)
                s0 = w[0] * gbuf[g0 + 0, sl] + w[1] * gbuf[g0 + 1, sl] + w[2] * gbuf[g0 + 2, sl]
                s1 = w[3] * gbuf[g0 + 3, sl] + w[4] * gbuf[g0 + 4, sl] + w[5] * gbuf[g0 + 5, sl]
                s2 = w[6] * gbuf[g0 + 6, sl] + w[7] * gbuf[g0 + 7, sl] + w[8] * gbuf[g0 + 8, sl]
                obuf[fb, sl] = e[0] * s0 + e[1] * s1 + e[2] * s2
                obuf[fb, pl.ds(D + t * 16, 16)] = e[3] * s0 + e[4] * s1 + e[5] * s2
        pltpu.sync_copy(obuf, faces2.at[pl.ds(c * FCH, FCH)])
        return carry

    lax.fori_loop(0, FNCH, chunk, 0)


def _lap_body(xt, lidx, lw, lap, idx_v, lw_v, gbuf, obuf, sem):
    c0 = _wid() * LNCH

    def chunk(ci, carry):
        c = c0 + ci
        pltpu.sync_copy(lidx.at[c], idx_v)
        pltpu.sync_copy(lw.at[c], lw_v)
        pltpu.async_copy(xt.at[idx_v], gbuf, sem).wait()
        for vb in range(LCH):
            g0 = vb * 7
            wb = min(g0, LCH * 7 - 16)
            wv = lw_v[pl.ds(wb, 16)]
            w = [wv[g0 - wb + j] for j in range(7)]
            for t in range(D // 16):
                sl = pl.ds(t * 16, 16)
                acc = w[0] * gbuf[g0 + 0, sl]
                for j in range(1, 7):
                    acc = acc + w[j] * gbuf[g0 + j, sl]
                obuf[vb, sl] = acc
        pltpu.sync_copy(obuf, lap.at[pl.ds(c * LCH, LCH)])
        return carry

    lax.fori_loop(0, LNCH, chunk, 0)


def _f2v_body(faces2, fidx, fw, gv, idx_v, fw_v, gbuf, obuf, sem):
    c0 = _wid() * VNCH

    def chunk(ci, carry):
        c = c0 + ci
        pltpu.sync_copy(fidx.at[c], idx_v)
        pltpu.sync_copy(fw.at[c], fw_v)
        pltpu.async_copy(faces2.at[idx_v], gbuf, sem).wait()
        for vb in range(VCH):
            g0 = vb * 6
            wb = min(g0, VCH * 6 - 16)
            wv = fw_v[pl.ds(wb, 16)]
            w = [wv[g0 - wb + j] for j in range(6)]
            for t in range(2 * D // 16):
                sl = pl.ds(t * 16, 16)
                acc = w[0] * gbuf[g0 + 0, sl]
                for j in range(1, 6):
                    acc = acc + w[j] * gbuf[g0 + j, sl]
                obuf[vb, sl] = acc
        pltpu.sync_copy(obuf, gv.at[pl.ds(c * VCH, VCH)])
        return carry

    lax.fori_loop(0, VNCH, chunk, 0)


def _sc_face(xt, gidx, gw, ewns):
    return pl.kernel(
        _face_body,
        out_type=jax.ShapeDtypeStruct((NF, 2 * D), jnp.float32),
        mesh=plsc.VectorSubcoreMesh(**_MESH),
        scratch_types=[
            pltpu.VMEM((FCH * 9,), jnp.int32),
            pltpu.VMEM((FCH * 9,), jnp.float32),
            pltpu.VMEM((FCH * 6,), jnp.float32),
            pltpu.VMEM((FCH * 9, D), jnp.float32),
            pltpu.VMEM((FCH, 2 * D), jnp.float32),
            pltpu.SemaphoreType.DMA,
        ],
    )(xt, gidx, gw, ewns)


def _sc_lap(xt, lidx, lw):
    return pl.kernel(
        _lap_body,
        out_type=jax.ShapeDtypeStruct((NVP, D), jnp.float32),
        mesh=plsc.VectorSubcoreMesh(**_MESH),
        scratch_types=[
            pltpu.VMEM((LCH * 7,), jnp.int32),
            pltpu.VMEM((LCH * 7,), jnp.float32),
            pltpu.VMEM((LCH * 7, D), jnp.float32),
            pltpu.VMEM((LCH, D), jnp.float32),
            pltpu.SemaphoreType.DMA,
        ],
    )(xt, lidx, lw)


def _sc_f2v(faces2, fidx, fw):
    return pl.kernel(
        _f2v_body,
        out_type=jax.ShapeDtypeStruct((NVP, 2 * D), jnp.float32),
        mesh=plsc.VectorSubcoreMesh(**_MESH),
        scratch_types=[
            pltpu.VMEM((VCH * 6,), jnp.int32),
            pltpu.VMEM((VCH * 6,), jnp.float32),
            pltpu.VMEM((VCH * 6, 2 * D), jnp.float32),
            pltpu.VMEM((VCH, 2 * D), jnp.float32),
            pltpu.SemaphoreType.DMA,
        ],
    )(faces2, fidx, fw)


_BLK = 512


def _combine_body(xt_ref, lap_ref, gv_ref, cs_ref, out_ref):
    cs = cs_ref[...]
    for b in range(BS):
        x = xt_ref[:, b * C:(b + 1) * C]
        l = lap_ref[:, b * C:(b + 1) * C]
        e = gv_ref[:, b * C:(b + 1) * C]
        n = gv_ref[:, D + b * C:D + (b + 1) * C]
        acc = (jnp.dot(x, cs[0:C], preferred_element_type=jnp.float32)
               + jnp.dot(l, cs[C:2 * C], preferred_element_type=jnp.float32)
               + jnp.dot(e, cs[2 * C:3 * C], preferred_element_type=jnp.float32)
               + jnp.dot(n, cs[3 * C:4 * C], preferred_element_type=jnp.float32))
        out_ref[b] = acc.T


def _tc_combine(xtp, lap, gv, cstack):
    return pl.pallas_call(
        _combine_body,
        grid=(NVP // _BLK,),
        in_specs=[
            pl.BlockSpec((_BLK, D), lambda i: (i, 0)),
            pl.BlockSpec((_BLK, D), lambda i: (i, 0)),
            pl.BlockSpec((_BLK, 2 * D), lambda i: (i, 0)),
            pl.BlockSpec((4 * C, C), lambda i: (0, 0)),
        ],
        out_specs=pl.BlockSpec((BS, C, _BLK), lambda i: (0, 0, i)),
        out_shape=jax.ShapeDtypeStruct((BS, C, NVP), jnp.float32),
    )(xtp, lap, gv, cstack)


def kernel(input, coeffs, G_rows, G_cols, G_vals, L_rows, L_cols, L_vals,
           F_rows, F_cols, F_vals, NS, EW):
    f32 = jnp.float32
    x = jnp.concatenate(
        [input, jnp.ones((BS, C, NV - NV_PREV), dtype=input.dtype)], axis=-1)
    xt = x.transpose(2, 0, 1).reshape(NV, D)
    xtp = jnp.concatenate([xt, jnp.zeros((NVP - NV, D), dtype=f32)], axis=0)

    gidx = G_cols.reshape(3, NF, 3).transpose(1, 0, 2).reshape(NF // FCH, FCH * 9)
    gw = G_vals.reshape(3, NF, 3).transpose(1, 0, 2).reshape(NF // FCH, FCH * 9)
    ewns = jnp.concatenate([EW, NS], axis=1).reshape(NF // FCH, FCH * 6)

    zpad = NVP - NV
    lidx = jnp.concatenate(
        [L_cols.reshape(NV, 7), jnp.zeros((zpad, 7), jnp.int32)], axis=0
    ).reshape(NVP // LCH, LCH * 7)
    lw = jnp.concatenate(
        [L_vals.reshape(NV, 7), jnp.zeros((zpad, 7), f32)], axis=0
    ).reshape(NVP // LCH, LCH * 7)

    fidx = jnp.concatenate(
        [F_cols.reshape(NV, 6), jnp.zeros((zpad, 6), jnp.int32)], axis=0
    ).reshape(NVP // VCH, VCH * 6)
    fw = jnp.concatenate(
        [F_vals.reshape(NV, 6), jnp.zeros((zpad, 6), f32)], axis=0
    ).reshape(NVP // VCH, VCH * 6)

    cstack = coeffs.reshape(C, 4, C).transpose(1, 0, 2).reshape(4 * C, C)

    faces2 = _sc_face(xtp, gidx, gw, ewns)
    lap = _sc_lap(xtp, lidx, lw)
    gv = _sc_f2v(faces2, fidx, fw)
    out = _tc_combine(xtp, lap, gv, cstack)
    return out[:, :, :NV]


# speedup vs baseline: 69.3530x; 1.6896x over previous
import functools

import jax
import jax.numpy as jnp
from jax import lax
from jax.experimental import pallas as pl
from jax.experimental.pallas import tpu as pltpu
from jax.experimental.pallas import tpu_sc as plsc

NV = 40962
NV_PREV = 10242
NF = 81920
BS = 2
C = 128
D = BS * C

_NC, _NSC = 2, 16
NW = _NC * _NSC
NVP = 41472
VPW = NVP // NW

FCH = 8
FNCH = (NF // NW) // FCH
LCH = 8
LNCH = VPW // LCH
VCH = 8
VNCH = VPW // VCH

_MESH = dict(core_axis_name="c", subcore_axis_name="s",
             num_cores=_NC, num_subcores=_NSC)


def _wid():
    return lax.axis_index("s") * _NC + lax.axis_index("c")


def _make_pass_body(rch, nch, compute_rows):
    def body(table, idx_hbm, w_hbm, out_hbm,
             idx0, idx1, wb0, wb1, gb0, gb1, ob0, ob1,
             smi0, smi1, smw0, smw1, smg0, smg1, smo0, smo1):
        idxb = (idx0, idx1)
        wb = (wb0, wb1)
        gb = (gb0, gb1)
        ob = (ob0, ob1)
        smi = (smi0, smi1)
        smw = (smw0, smw1)
        smg = (smg0, smg1)
        smo = (smo0, smo1)
        c0 = _wid() * nch

        pltpu.sync_copy(idx_hbm.at[c0], idx0)
        pltpu.async_copy(idx_hbm.at[c0 + 1], idx1, smi1)
        pltpu.async_copy(w_hbm.at[c0], wb0, smw0)
        pltpu.async_copy(table.at[idx0], gb0, smg0)

        def pair(p, carry):
            for b in range(2):
                ci = 2 * p + b
                c = c0 + ci
                s, s1 = b, 1 - b
                pltpu.make_async_copy(table.at[idxb[s]], gb[s], smg[s]).wait()

                @pl.when(ci + 2 < nch)
                def _():
                    pltpu.async_copy(idx_hbm.at[c + 2], idxb[s], smi[s])

                @pl.when(ci + 1 < nch)
                def _():
                    pltpu.make_async_copy(idx_hbm.at[c + 1], idxb[s1], smi[s1]).wait()
                    pltpu.async_copy(w_hbm.at[c + 1], wb[s1], smw[s1])
                    pltpu.async_copy(table.at[idxb[s1]], gb[s1], smg[s1])

                pltpu.make_async_copy(w_hbm.at[c], wb[s], smw[s]).wait()

                @pl.when(ci >= 2)
                def _():
                    pltpu.make_async_copy(
                        ob[s], out_hbm.at[pl.ds(c * rch, rch)], smo[s]).wait()

                compute_rows(gb[s], wb[s], ob[s])
                pltpu.async_copy(ob[s], out_hbm.at[pl.ds(c * rch, rch)], smo[s])
            return carry

        lax.fori_loop(0, nch // 2, pair, 0)
        pltpu.make_async_copy(ob[0], out_hbm.at[pl.ds(c0 * rch, rch)], smo[0]).wait()
        pltpu.make_async_copy(ob[1], out_hbm.at[pl.ds(c0 * rch, rch)], smo[1]).wait()

    return body


def _face_rows(gb, wvb, ob):
    def face(fb, carry):
        g0 = fb * 9
        wv = wvb[pl.ds(fb * 32, 16)]
        ev = wvb[pl.ds(fb * 32 + 16, 16)]
        w = [wv[j] for j in range(9)]
        e = [ev[j] for j in range(6)]
        for t in range(D // 16):
            sl = pl.ds(t * 16, 16)
            s0 = w[0] * gb[g0 + 0, sl] + w[1] * gb[g0 + 1, sl] + w[2] * gb[g0 + 2, sl]
            s1 = w[3] * gb[g0 + 3, sl] + w[4] * gb[g0 + 4, sl] + w[5] * gb[g0 + 5, sl]
            s2 = w[6] * gb[g0 + 6, sl] + w[7] * gb[g0 + 7, sl] + w[8] * gb[g0 + 8, sl]
            ob[fb, sl] = e[0] * s0 + e[1] * s1 + e[2] * s2
            ob[fb, pl.ds(D + t * 16, 16)] = e[3] * s0 + e[4] * s1 + e[5] * s2
        return carry

    lax.fori_loop(0, FCH, face, 0)


def _lap_rows(gb, wvb, ob):
    def vert(vb, carry):
        g0 = vb * 7
        wv = wvb[pl.ds(vb * 16, 16)]
        w = [wv[j] for j in range(7)]
        for t in range(D // 16):
            sl = pl.ds(t * 16, 16)
            acc = w[0] * gb[g0 + 0, sl]
            for j in range(1, 7):
                acc = acc + w[j] * gb[g0 + j, sl]
            ob[vb, sl] = acc
        return carry

    lax.fori_loop(0, LCH, vert, 0)


def _f2v_rows(gb, wvb, ob):
    def vert(vb, carry):
        g0 = vb * 6
        wv = wvb[pl.ds(vb * 16, 16)]
        w = [wv[j] for j in range(6)]
        for t in range(2 * D // 16):
            sl = pl.ds(t * 16, 16)
            acc = w[0] * gb[g0 + 0, sl]
            for j in range(1, 6):
                acc = acc + w[j] * gb[g0 + j, sl]
            ob[vb, sl] = acc
        return carry

    lax.fori_loop(0, VCH, vert, 0)


def _pass_scratch(nidx, wlen, outd, rch):
    return [
        pltpu.VMEM((nidx,), jnp.int32),
        pltpu.VMEM((nidx,), jnp.int32),
        pltpu.VMEM((wlen,), jnp.float32),
        pltpu.VMEM((wlen,), jnp.float32),
        pltpu.VMEM((nidx, D), jnp.float32),
        pltpu.VMEM((nidx, D), jnp.float32),
        pltpu.VMEM((rch, outd), jnp.float32),
        pltpu.VMEM((rch, outd), jnp.float32),
    ] + [pltpu.SemaphoreType.DMA] * 8


def _sc_face(xt, gidx, gw):
    scr = _pass_scratch(FCH * 9, FCH * 32, 2 * D, FCH)
    return pl.kernel(
        _make_pass_body(FCH, FNCH, _face_rows),
        out_type=jax.ShapeDtypeStruct((NF, 2 * D), jnp.float32),
        mesh=plsc.VectorSubcoreMesh(**_MESH),
        scratch_types=scr,
    )(xt, gidx, gw)


def _sc_lap(xt, lidx, lw):
    scr = _pass_scratch(LCH * 7, LCH * 16, D, LCH)
    return pl.kernel(
        _make_pass_body(LCH, LNCH, _lap_rows),
        out_type=jax.ShapeDtypeStruct((NVP, D), jnp.float32),
        mesh=plsc.VectorSubcoreMesh(**_MESH),
        scratch_types=scr,
    )(xt, lidx, lw)


def _sc_f2v(faces2, fidx, fw):
    scr = [
        pltpu.VMEM((VCH * 6,), jnp.int32),
        pltpu.VMEM((VCH * 6,), jnp.int32),
        pltpu.VMEM((VCH * 16,), jnp.float32),
        pltpu.VMEM((VCH * 16,), jnp.float32),
        pltpu.VMEM((VCH * 6, 2 * D), jnp.float32),
        pltpu.VMEM((VCH * 6, 2 * D), jnp.float32),
        pltpu.VMEM((VCH, 2 * D), jnp.float32),
        pltpu.VMEM((VCH, 2 * D), jnp.float32),
    ] + [pltpu.SemaphoreType.DMA] * 8
    return pl.kernel(
        _make_pass_body(VCH, VNCH, _f2v_rows),
        out_type=jax.ShapeDtypeStruct((NVP, 2 * D), jnp.float32),
        mesh=plsc.VectorSubcoreMesh(**_MESH),
        scratch_types=scr,
    )(faces2, fidx, fw)


_BLK = 512


def _combine_body(xt_ref, lap_ref, gv_ref, cs_ref, out_ref):
    cs = cs_ref[...]
    for b in range(BS):
        x = xt_ref[:, b * C:(b + 1) * C]
        l = lap_ref[:, b * C:(b + 1) * C]
        e = gv_ref[:, b * C:(b + 1) * C]
        n = gv_ref[:, D + b * C:D + (b + 1) * C]
        acc = (jnp.dot(x, cs[0:C], preferred_element_type=jnp.float32)
               + jnp.dot(l, cs[C:2 * C], preferred_element_type=jnp.float32)
               + jnp.dot(e, cs[2 * C:3 * C], preferred_element_type=jnp.float32)
               + jnp.dot(n, cs[3 * C:4 * C], preferred_element_type=jnp.float32))
        out_ref[b] = acc.T


def _tc_combine(xtp, lap, gv, cstack):
    return pl.pallas_call(
        _combine_body,
        grid=(NVP // _BLK,),
        in_specs=[
            pl.BlockSpec((_BLK, D), lambda i: (i, 0)),
            pl.BlockSpec((_BLK, D), lambda i: (i, 0)),
            pl.BlockSpec((_BLK, 2 * D), lambda i: (i, 0)),
            pl.BlockSpec((4 * C, C), lambda i: (0, 0)),
        ],
        out_specs=pl.BlockSpec((BS, C, _BLK), lambda i: (0, 0, i)),
        out_shape=jax.ShapeDtypeStruct((BS, C, NVP), jnp.float32),
    )(xtp, lap, gv, cstack)


def kernel(input, coeffs, G_rows, G_cols, G_vals, L_rows, L_cols, L_vals,
           F_rows, F_cols, F_vals, NS, EW):
    f32 = jnp.float32
    x = jnp.concatenate(
        [input, jnp.ones((BS, C, NV - NV_PREV), dtype=input.dtype)], axis=-1)
    xt = x.transpose(2, 0, 1).reshape(NV, D)
    xtp = jnp.concatenate([xt, jnp.zeros((NVP - NV, D), dtype=f32)], axis=0)

    gcols9 = G_cols.reshape(3, NF, 3).transpose(1, 0, 2).reshape(NF, 9)
    gvals9 = G_vals.reshape(3, NF, 3).transpose(1, 0, 2).reshape(NF, 9)
    gidx = gcols9.reshape(NF // FCH, FCH * 9)
    gw = jnp.concatenate(
        [gvals9, jnp.zeros((NF, 7), f32), EW, NS, jnp.zeros((NF, 10), f32)],
        axis=1).reshape(NF // FCH, FCH * 32)

    zpad = NVP - NV
    lidx = jnp.concatenate(
        [L_cols.reshape(NV, 7), jnp.zeros((zpad, 7), jnp.int32)], axis=0
    ).reshape(NVP // LCH, LCH * 7)
    lw = jnp.concatenate(
        [L_vals.reshape(NV, 7), jnp.zeros((NV, 9), f32)], axis=1)
    lw = jnp.concatenate([lw, jnp.zeros((zpad, 16), f32)], axis=0
                         ).reshape(NVP // LCH, LCH * 16)

    fidx = jnp.concatenate(
        [F_cols.reshape(NV, 6), jnp.zeros((zpad, 6), jnp.int32)], axis=0
    ).reshape(NVP // VCH, VCH * 6)
    fw = jnp.concatenate(
        [F_vals.reshape(NV, 6), jnp.zeros((NV, 10), f32)], axis=1)
    fw = jnp.concatenate([fw, jnp.zeros((zpad, 16), f32)], axis=0
                         ).reshape(NVP // VCH, VCH * 16)

    cstack = coeffs.reshape(C, 4, C).transpose(1, 0, 2).reshape(4 * C, C)

    faces2 = _sc_face(xtp, gidx, gw)
    lap = _sc_lap(xtp, lidx, lw)
    gv = _sc_f2v(faces2, fidx, fw)
    out = _tc_combine(xtp, lap, gv, cstack)
    return out[:, :, :NV]
